# baseline (speedup 1.0000x reference)
import functools

import jax
import jax.numpy as jnp
from jax import lax
from jax.experimental import pallas as pl
from jax.experimental.pallas import tpu as pltpu
from jax.experimental.pallas import tpu_sc as plsc

NC = 2
NS = 16
LANES = 16
NW = NC * NS

_PARAMS = pltpu.CompilerParams(
    needs_layout_passes=False, use_tc_tiling_on_sc=False)
_MESH = dict(core_axis_name="c", subcore_axis_name="s")


def _make_multi_kernel(B, NMF, L, V, D):
    RB = B // NW
    NT = RB * NMF
    NB = NT // LANES
    SEC = LANES * L

    @functools.partial(
        pl.kernel,
        out_type=jax.ShapeDtypeStruct((B * NMF, D), jnp.float32),
        mesh=plsc.VectorSubcoreMesh(**_MESH),
        compiler_params=_PARAMS,
        scratch_types=[
            [pltpu.VMEM((SEC,), jnp.int32)] * 2,
            [pltpu.VMEM((SEC,), jnp.float32)] * 2,
            pltpu.VMEM((NT,), jnp.int32),
            [pltpu.VMEM((SEC, D), jnp.float32)] * 2,
            [pltpu.VMEM((LANES, D), jnp.float32)] * 2,
            [pltpu.SemaphoreType.DMA] * 6,
        ],
    )
    def k(tm_hbm, xm_hbm, vals_hbm, len_hbm, out_hbm,
          midx_r, vals_r, len_v, stage_r, pool_r, sems):
        sem_in = sems[0:2]
        sem_st = sems[2:4]
        sem_out = sems[4:6]
        wid = lax.axis_index("s") * NC + lax.axis_index("c")
        base = pl.multiple_of(wid * RB, RB)
        fbase = base * NMF * L
        iota = lax.iota(jnp.int32, LANES)
        tb = iota * L

        def issue_in(t, p):
            tc = jnp.minimum(t, NB - 2 + p)
            off = pl.multiple_of(fbase + tc * SEC, SEC)
            pltpu.async_copy(xm_hbm.at[pl.ds(off, SEC)], midx_r[p], sem_in[p])
            pltpu.async_copy(vals_hbm.at[pl.ds(off, SEC)], vals_r[p],
                             sem_in[p])

        def wait_in(p):
            pltpu.make_async_copy(xm_hbm.at[pl.ds(0, SEC)], midx_r[p],
                                  sem_in[p]).wait()
            pltpu.make_async_copy(vals_hbm.at[pl.ds(0, SEC)], vals_r[p],
                                  sem_in[p]).wait()

        def issue_stream(p):
            pltpu.async_copy(tm_hbm.at[midx_r[p]], stage_r[p], sem_st[p])

        def wait_stream(p):
            pltpu.make_async_copy(tm_hbm.at[pl.ds(0, SEC)], stage_r[p],
                                  sem_st[p]).wait()

        def issue_out(t, p):
            pltpu.async_copy(
                pool_r[p],
                out_hbm.at[pl.ds(pl.multiple_of(
                    base * NMF + t * LANES, LANES), LANES)],
                sem_out[p])

        def wait_out(p):
            pltpu.make_async_copy(pool_r[p], out_hbm.at[pl.ds(0, LANES)],
                                  sem_out[p]).wait()

        def compute(t, p):
            tglob = t * LANES + iota
            pad = tglob % NMF * V

            def l_body(l, acc, pad=pad):
                jv = tb + l
                iv = plsc.load_gather(midx_r[p], [jv])
                wv = plsc.load_gather(vals_r[p], [jv])
                wv = jnp.where(iv != pad, wv, 0.0)
                dvec = jnp.zeros((LANES,), jnp.int32)
                out = []
                for d in range(D):
                    gv = plsc.load_gather(stage_r[p], [jv, dvec])
                    dvec = dvec + 1
                    out.append(acc[d] + wv * gv)
                return tuple(out)

            acc = lax.fori_loop(
                0, L, l_body,
                tuple(jnp.zeros((LANES,), jnp.float32) for _ in range(D)))

            lv = plsc.load_gather(len_v, [tglob]).astype(jnp.float32)
            inv = 1.0 / jnp.maximum(lv, 1.0)
            wait_out(p)
            dvec = jnp.zeros((LANES,), jnp.int32)
            for d in range(D):
                plsc.store_scatter(pool_r[p], [iota, dvec], acc[d] * inv)
                dvec = dvec + 1
            issue_out(t, p)

        pltpu.sync_copy(len_hbm.at[pl.ds(base * NMF, NT)], len_v)
        issue_in(0, 0)
        issue_in(1, 1)
        issue_out(0, 0)
        issue_out(1, 1)

        def pipe_body(i, carry):
            t0 = pl.multiple_of(i * 2, 2)
            t1 = t0 + 1
            wait_in(0)
            issue_stream(0)
            wait_in(1)
            issue_stream(1)
            wait_stream(0)
            compute(t0, 0)
            issue_in(t0 + 2, 0)
            wait_stream(1)
            compute(t1, 1)
            issue_in(t1 + 2, 1)
            return carry

        lax.fori_loop(0, NB // 2, pipe_body, 0)

        wait_in(0)
        wait_in(1)
        wait_out(0)
        wait_out(1)

    return k


def _make_single_kernel(B, NSF, V, D):
    RB = B // NW
    PR = 2
    NSTR = RB // PR

    @functools.partial(
        pl.kernel,
        out_type=jax.ShapeDtypeStruct((B * NSF, D), jnp.float32),
        mesh=plsc.VectorSubcoreMesh(**_MESH),
        compiler_params=_PARAMS,
        scratch_types=[
            pltpu.VMEM((RB * NSF,), jnp.int32),
            pltpu.VMEM((RB * NSF, D), jnp.float32),
            pltpu.SemaphoreType.DMA,
        ],
    )
    def k(ts_hbm, xs_hbm, out_hbm, sidx_v, rows_v, sem):
        wid = lax.axis_index("s") * NC + lax.axis_index("c")
        base = pl.multiple_of(wid * RB, RB)
        pltpu.sync_copy(xs_hbm.at[pl.ds(base * NSF, RB * NSF)], sidx_v)
        copies = [
            pltpu.async_copy(
                ts_hbm.at[sidx_v.at[pl.ds(p * PR * NSF, PR * NSF)]],
                rows_v.at[pl.ds(p * PR * NSF, PR * NSF)], sem)
            for p in range(NSTR)
        ]
        for c in copies:
            c.wait()
        pltpu.sync_copy(rows_v, out_hbm.at[pl.ds(base * NSF, RB * NSF)])

    return k


def kernel(x_single, x_multi, x_multi_vals, x_multi_lens,
           single_tables, multi_tables):
    NSF, V, D = single_tables.shape
    NMF = multi_tables.shape[0]
    B, _, L = x_multi.shape

    idx_s = x_single.astype(jnp.int32) + jnp.arange(NSF, dtype=jnp.int32) * V
    idx_m = (x_multi.astype(jnp.int32)
             + (jnp.arange(NMF, dtype=jnp.int32) * V)[None, :, None])

    km = _make_multi_kernel(B, NMF, L, V, D)
    out_m = km(multi_tables.reshape(NMF * V, D),
               idx_m.reshape(B * NMF * L),
               x_multi_vals.reshape(B * NMF * L),
               x_multi_lens.astype(jnp.int32).reshape(B * NMF))
    ks = _make_single_kernel(B, NSF, V, D)
    out_s = ks(single_tables.reshape(NSF * V, D), idx_s.reshape(B * NSF))
    return jnp.concatenate(
        [out_s.reshape(B, NSF * D), out_m.reshape(B, NMF * D)], axis=1)
